# baseline (speedup 1.0000x reference)
import jax
import jax.numpy as jnp
from jax import lax
from jax.experimental import pallas as pl
from jax.experimental.pallas import tpu as pltpu

N = 1024
IN_DIM = 512
HID = 256
OUT = 128
BLK = 128
NB = N // BLK


def _leaky(v, slope=0.25):
    return jnp.where(v >= 0, v, slope * v)


def _encoder_body(x_ref, adj_ref, w1_ref, b1_ref, w2_ref, b2_ref, wg_ref,
                  a1_ref, a2_ref, out_ref,
                  adj_s, s_s, x1_s, h_s, ha1_s, ha2_s):
    p = pl.program_id(0)
    j = pl.program_id(1)
    f32 = jnp.float32
    rows = pl.ds(j * BLK, BLK)

    @pl.when(p == 0)
    def _phase0():
        adj_blk = adj_ref[...]
        adj_s[rows, :] = adj_blk

        @pl.when(j == 0)
        def _():
            s_s[...] = jnp.dot(x_ref[...], w1_ref[...],
                               preferred_element_type=f32)

        x1_s[rows, :] = _leaky(
            jnp.dot(adj_blk, s_s[...], preferred_element_type=f32)
            + b1_ref[...])

    @pl.when(p == 1)
    def _phase1():
        @pl.when(j == 0)
        def _():
            s_s[...] = jnp.dot(x1_s[...], w2_ref[...],
                               preferred_element_type=f32)

        x2_blk = _leaky(
            jnp.dot(adj_s[rows, :], s_s[...], preferred_element_type=f32)
            + b2_ref[...])
        h_blk = jnp.dot(x2_blk, wg_ref[...], preferred_element_type=f32)
        h_s[rows, :] = h_blk
        ha1_s[rows, :] = jnp.sum(h_blk * a1_ref[...], axis=1, keepdims=True)

    @pl.when(p == 2)
    def _phase2():
        @pl.when(j == 0)
        def _():
            ha2_s[...] = lax.dot_general(
                a2_ref[...], h_s[...], (((1,), (1,)), ((), ())),
                preferred_element_type=f32)

        e = _leaky(ha1_s[rows, :] + ha2_s[...])
        att = jnp.where(adj_s[rows, :] > 0, e, jnp.float32(-1e12))
        att = jnp.exp(att - jnp.max(att, axis=1, keepdims=True))
        acc = jnp.dot(att, h_s[...], preferred_element_type=f32)
        scale = 1.0 / jnp.sum(att, axis=1, keepdims=True)
        out_ref[...] = _leaky(acc * scale)


def kernel(x, adj, W1, b1, W2, b2, Wg, a):
    full = lambda shape: pl.BlockSpec(shape, lambda p, j: (0,) * len(shape))
    out = pl.pallas_call(
        _encoder_body,
        grid=(3, NB),
        in_specs=[
            full((N, IN_DIM)),
            pl.BlockSpec((BLK, N),
                         lambda p, j: (jnp.where(p == 0, j, NB - 1), 0)),
            full((IN_DIM, HID)),
            full((1, HID)),
            full((HID, HID)),
            full((1, HID)),
            full((HID, OUT)),
            full((1, OUT)),
            full((1, OUT)),
        ],
        out_specs=pl.BlockSpec((BLK, OUT), lambda p, j: (j, 0)),
        out_shape=jax.ShapeDtypeStruct((N, OUT), jnp.float32),
        scratch_shapes=[
            pltpu.VMEM((N, N), jnp.float32),
            pltpu.VMEM((N, HID), jnp.float32),
            pltpu.VMEM((N, HID), jnp.float32),
            pltpu.VMEM((N, OUT), jnp.float32),
            pltpu.VMEM((N, 1), jnp.float32),
            pltpu.VMEM((1, N), jnp.float32),
        ],
        compiler_params=pltpu.CompilerParams(
            dimension_semantics=("arbitrary", "arbitrary")),
    )(x, adj, W1, b1.reshape(1, HID), W2, b2.reshape(1, HID), Wg,
      a[:OUT].reshape(1, OUT), a[OUT:].reshape(1, OUT))
    return out[:, : OUT // 2], out[:, OUT // 2:]


# speedup vs baseline: 1.5028x; 1.5028x over previous
import jax
import jax.numpy as jnp
from jax import lax
from jax.experimental import pallas as pl
from jax.experimental.pallas import tpu as pltpu

N = 1024
IN_DIM = 512
HID = 256
OUT = 128
NCHUNK = 4
CHUNK = N // NCHUNK


def _leaky(v, slope=0.25):
    return jnp.where(v >= 0, v, slope * v)


def _encoder_body(x_hbm, adj_hbm, w1_ref, b1_ref, w2_ref, b2_ref, wg_ref,
                  a_ref, mu_ref, lv_ref,
                  x_v, adj_v, sem_x, sem_adj):
    f32 = jnp.float32
    x_cp = pltpu.make_async_copy(x_hbm, x_v, sem_x)
    x_cp.start()
    adj_cps = []
    for c in range(NCHUNK):
        rows = pl.ds(c * CHUNK, CHUNK)
        cp = pltpu.make_async_copy(adj_hbm.at[rows, :], adj_v.at[rows, :],
                                   sem_adj)
        cp.start()
        adj_cps.append(cp)

    x_cp.wait()
    s1 = jnp.dot(x_v[...], w1_ref[...], preferred_element_type=f32)

    x1_parts = []
    for c in range(NCHUNK):
        adj_cps[c].wait()
        rows = pl.ds(c * CHUNK, CHUNK)
        x1_parts.append(_leaky(
            jnp.dot(adj_v[rows, :], s1, preferred_element_type=f32)
            + b1_ref[...]))
    x1 = jnp.concatenate(x1_parts, axis=0)

    adj = adj_v[...]
    s2 = jnp.dot(x1, w2_ref[...], preferred_element_type=f32)
    x2 = _leaky(jnp.dot(adj, s2, preferred_element_type=f32) + b2_ref[...])
    h = jnp.dot(x2, wg_ref[...], preferred_element_type=f32)
    a1 = a_ref[:, :OUT]
    a2 = a_ref[:, OUT:]
    ha1 = jnp.sum(h * a1, axis=1, keepdims=True)
    ha2 = lax.dot_general(a2, h, (((1,), (1,)), ((), ())),
                          preferred_element_type=f32)
    e = _leaky(ha1 + ha2)
    att = jnp.where(adj > 0, e, jnp.float32(-1e12))
    att = jnp.exp(att - jnp.max(att, axis=1, keepdims=True))
    acc = jnp.dot(att, h, preferred_element_type=f32)
    out = _leaky(acc * (1.0 / jnp.sum(att, axis=1, keepdims=True)))
    mu_ref[...] = out[:, : OUT // 2]
    lv_ref[...] = out[:, OUT // 2:]


def kernel(x, adj, W1, b1, W2, b2, Wg, a):
    hbm = pl.BlockSpec(memory_space=pltpu.MemorySpace.HBM)
    vmem = pl.BlockSpec(memory_space=pltpu.MemorySpace.VMEM)
    mu, lv = pl.pallas_call(
        _encoder_body,
        in_specs=[hbm, hbm] + [vmem] * 6,
        out_specs=(vmem, vmem),
        out_shape=(jax.ShapeDtypeStruct((N, OUT // 2), jnp.float32),
                   jax.ShapeDtypeStruct((N, OUT // 2), jnp.float32)),
        scratch_shapes=[
            pltpu.MemorySpace.VMEM((N, IN_DIM), jnp.float32),
            pltpu.MemorySpace.VMEM((N, N), jnp.float32),
            pltpu.SemaphoreType.DMA,
            pltpu.SemaphoreType.DMA,
        ],
    )(x, adj, W1, b1.reshape(1, HID), W2, b2.reshape(1, HID), Wg,
      a.reshape(1, 2 * OUT))
    return mu, lv


# speedup vs baseline: 2.0472x; 1.3623x over previous
import jax
import jax.numpy as jnp
from jax import lax
from jax.experimental import pallas as pl
from jax.experimental.pallas import tpu as pltpu

N = 1024
IN_DIM = 512
HID = 256
OUT = 128
NCHUNK = 4
CHUNK = N // NCHUNK


def _leaky(v, slope=0.25):
    return jnp.where(v >= 0, v, slope * v)


def _encoder_body(x_hbm, adj_hbm, w1_ref, b1_ref, w2_ref, b2_ref, wg_ref,
                  a_ref, mu_ref, lv_ref,
                  x_v, adj_v, sem_x, sem_adj):
    f32 = jnp.float32
    x_cp = pltpu.make_async_copy(x_hbm, x_v, sem_x)
    x_cp.start()
    adj_cps = []
    for c in range(NCHUNK):
        rows = pl.ds(c * CHUNK, CHUNK)
        cp = pltpu.make_async_copy(adj_hbm.at[rows, :], adj_v.at[rows, :],
                                   sem_adj.at[c])
        cp.start()
        adj_cps.append(cp)

    x_cp.wait()
    s1 = jnp.dot(x_v[...], w1_ref[...], preferred_element_type=f32)

    x1_parts = []
    for c in range(NCHUNK):
        adj_cps[c].wait()
        rows = pl.ds(c * CHUNK, CHUNK)
        x1_parts.append(_leaky(
            jnp.dot(adj_v[rows, :], s1, preferred_element_type=f32)
            + b1_ref[...]))
    x1 = jnp.concatenate(x1_parts, axis=0)

    adj = adj_v[...]
    s2 = jnp.dot(x1, w2_ref[...], preferred_element_type=f32)
    x2 = _leaky(jnp.dot(adj, s2, preferred_element_type=f32) + b2_ref[...])
    h = jnp.dot(x2, wg_ref[...], preferred_element_type=f32)
    a1 = a_ref[:, :OUT]
    a2 = a_ref[:, OUT:]
    ha1 = jnp.sum(h * a1, axis=1, keepdims=True)
    ha2 = lax.dot_general(a2, h, (((1,), (1,)), ((), ())),
                          preferred_element_type=f32)
    e = _leaky(ha1 + ha2)
    att = jnp.where(adj > 0, e, jnp.float32(-1e12))
    att = jnp.exp(att - jnp.max(att, axis=1, keepdims=True))
    acc = jnp.dot(att, h, preferred_element_type=f32)
    out = _leaky(acc * (1.0 / jnp.sum(att, axis=1, keepdims=True)))
    out_t = out.T
    mu_ref[...] = out_t[: OUT // 2, :]
    lv_ref[...] = out_t[OUT // 2:, :]


def kernel(x, adj, W1, b1, W2, b2, Wg, a):
    hbm = pl.BlockSpec(memory_space=pltpu.MemorySpace.HBM)
    vmem = pl.BlockSpec(memory_space=pltpu.MemorySpace.VMEM)
    mu_t, lv_t = pl.pallas_call(
        _encoder_body,
        in_specs=[hbm, hbm] + [vmem] * 6,
        out_specs=(vmem, vmem),
        out_shape=(jax.ShapeDtypeStruct((OUT // 2, N), jnp.float32),
                   jax.ShapeDtypeStruct((OUT // 2, N), jnp.float32)),
        scratch_shapes=[
            pltpu.MemorySpace.VMEM((N, IN_DIM), jnp.float32),
            pltpu.MemorySpace.VMEM((N, N), jnp.float32),
            pltpu.SemaphoreType.DMA,
            pltpu.SemaphoreType.DMA((NCHUNK,)),
        ],
    )(x, adj, W1, b1.reshape(1, HID), W2, b2.reshape(1, HID), Wg,
      a.reshape(1, 2 * OUT))
    return mu_t.T, lv_t.T
